# baseline (speedup 1.0000x reference)
import functools

import jax
import jax.numpy as jnp
from jax import lax
from jax.experimental import pallas as pl
from jax.experimental.pallas import tpu as pltpu
from jax.experimental.pallas import tpu_sc as plsc

BF = jnp.bfloat16
F32 = jnp.float32

NK = 8


def _cbsq_body(cbf_ref, out_ref):
    cbf = cbf_ref[...]
    out_ref[...] = jnp.sum(cbf * cbf, axis=1).reshape(1, -1)


def _enc_vq_body(x_ref, wb_ref, be_ref, cbb_ref, cbsq_ref, out_ref):
    xb = x_ref[...].astype(BF)
    z_e = lax.dot_general(xb, wb_ref[...], (((1,), (0,)), ((), ())),
                          preferred_element_type=F32) + be_ref[...]
    ze_sq = jnp.sum(z_e * z_e, axis=1, keepdims=True)
    zeb = z_e.astype(BF)
    K = cbb_ref.shape[0]
    KT = K // NK
    rmin = None
    ridx = None
    for j in range(NK):
        cb_j = cbb_ref[pl.ds(j * KT, KT), :]
        dots = lax.dot_general(zeb, cb_j, (((1,), (1,)), ((), ())),
                               preferred_element_type=F32)
        dist = ze_sq - 2.0 * dots + cbsq_ref[:, pl.ds(j * KT, KT)]
        vmin = jnp.min(dist, axis=1, keepdims=True)
        kidx = lax.broadcasted_iota(jnp.int32, dist.shape, 1)
        cand = jnp.where(dist == vmin, kidx, KT)
        lidx = jnp.min(cand, axis=1, keepdims=True) + j * KT
        if j == 0:
            rmin, ridx = vmin, lidx
        else:
            better = vmin < rmin
            ridx = jnp.where(better, lidx, ridx)
            rmin = jnp.where(better, vmin, rmin)
    out_ref[...] = ridx.reshape(1, 1, -1)


def _dec_body(zq_ref, wdb_ref, bd_ref, out_ref):
    zqb = zq_ref[...].astype(BF)
    out_ref[...] = lax.dot_general(
        zqb, wdb_ref[...], (((1,), (0,)), ((), ())),
        preferred_element_type=F32) + bd_ref[...]


def _sc_gather(codebook, idx):
    K, D = codebook.shape
    N = idx.shape[0]
    info = plsc.get_sparse_core_info()
    nw = info.num_cores * info.num_subcores
    n_per_w = N // nw
    mesh = plsc.VectorSubcoreMesh(core_axis_name="c", subcore_axis_name="s")

    @functools.partial(
        pl.kernel, mesh=mesh,
        out_type=jax.ShapeDtypeStruct((N, D), F32),
        scratch_types=[
            pltpu.VMEM((n_per_w,), jnp.int32),
            pltpu.VMEM((n_per_w, D), F32),
            pltpu.SemaphoreType.DMA,
        ],
    )
    def gather_k(table_hbm, idx_hbm, out_hbm, idx_v, rows_v, sem):
        wid = lax.axis_index("s") * info.num_cores + lax.axis_index("c")
        base = wid * n_per_w
        pltpu.sync_copy(idx_hbm.at[pl.ds(base, n_per_w)], idx_v)
        pltpu.async_copy(table_hbm.at[idx_v], rows_v, sem).wait()
        pltpu.sync_copy(rows_v, out_hbm.at[pl.ds(base, n_per_w)])

    return gather_k(codebook, idx)


def kernel(x, W_enc, b_enc, codebook, W_dec, b_dec):
    B, T, DIN = x.shape
    D = W_enc.shape[1]
    K = codebook.shape[0]
    N = B * T
    TM = 512
    x2 = x.reshape(N, DIN)

    cb_sq = pl.pallas_call(
        _cbsq_body,
        out_shape=jax.ShapeDtypeStruct((1, K), F32),
    )(codebook)

    z_ind3 = pl.pallas_call(
        _enc_vq_body,
        grid=(N // TM,),
        in_specs=[
            pl.BlockSpec((TM, DIN), lambda i: (i, 0)),
            pl.BlockSpec((DIN, D), lambda i: (0, 0)),
            pl.BlockSpec((1, D), lambda i: (0, 0)),
            pl.BlockSpec((K, D), lambda i: (0, 0)),
            pl.BlockSpec((1, K), lambda i: (0, 0)),
        ],
        out_specs=pl.BlockSpec((1, 1, TM), lambda i: (i, 0, 0)),
        out_shape=jax.ShapeDtypeStruct((N // TM, 1, TM), jnp.int32),
    )(x2, W_enc.astype(BF), b_enc.reshape(1, D), codebook.astype(BF), cb_sq)
    z_ind = z_ind3.reshape(N)

    z_q = _sc_gather(codebook, z_ind)

    TMD = 512
    x_rec = pl.pallas_call(
        _dec_body,
        grid=(N // TMD,),
        in_specs=[
            pl.BlockSpec((TMD, D), lambda i: (i, 0)),
            pl.BlockSpec((D, DIN), lambda i: (0, 0)),
            pl.BlockSpec((1, DIN), lambda i: (0, 0)),
        ],
        out_specs=pl.BlockSpec((TMD, DIN), lambda i: (i, 0)),
        out_shape=jax.ShapeDtypeStruct((N, DIN), F32),
    )(z_q, W_dec.astype(BF), b_dec.reshape(1, DIN))

    return (x_rec.reshape(B, T, DIN), z_q.reshape(B, T, D),
            z_ind.reshape(B, T))


# speedup vs baseline: 1.1667x; 1.1667x over previous
import functools

import jax
import jax.numpy as jnp
from jax import lax
from jax.experimental import pallas as pl
from jax.experimental.pallas import tpu as pltpu
from jax.experimental.pallas import tpu_sc as plsc

BF = jnp.bfloat16
F32 = jnp.float32

NK = 8


def _cbsq_body(cbf_ref, out_ref):
    cbf = cbf_ref[...]
    out_ref[...] = jnp.sum(cbf * cbf, axis=1).reshape(1, -1)


def _enc_vq_body(x_ref, wb_ref, be_ref, cbb_ref, cbsq_ref, kidx_ref, out_ref):
    xb = x_ref[...].astype(BF)
    z_e = lax.dot_general(xb, wb_ref[...], (((1,), (0,)), ((), ())),
                          preferred_element_type=F32) + be_ref[...]
    ze_sq = jnp.sum(z_e * z_e, axis=1, keepdims=True)
    zeb = z_e.astype(BF)
    K = cbb_ref.shape[0]
    dots = lax.dot_general(zeb, cbb_ref[...], (((1,), (1,)), ((), ())),
                           preferred_element_type=F32)
    dist = ze_sq - 2.0 * dots + cbsq_ref[...]
    vmin = jnp.min(dist, axis=1, keepdims=True)
    cand = jnp.where(dist == vmin, kidx_ref[...], float(K))
    idxf = jnp.min(cand, axis=1)
    out_ref[...] = idxf.astype(jnp.int32).reshape(1, 1, -1)


def _dec_body(zq_ref, wdb_ref, bd_ref, out_ref):
    zqb = zq_ref[...].astype(BF)
    out_ref[...] = lax.dot_general(
        zqb, wdb_ref[...], (((1,), (0,)), ((), ())),
        preferred_element_type=F32) + bd_ref[...]


def _sc_gather(codebook, idx):
    K, D = codebook.shape
    N = idx.shape[0]
    info = plsc.get_sparse_core_info()
    nw = info.num_cores * info.num_subcores
    n_per_w = N // nw
    mesh = plsc.VectorSubcoreMesh(core_axis_name="c", subcore_axis_name="s")

    @functools.partial(
        pl.kernel, mesh=mesh,
        out_type=jax.ShapeDtypeStruct((N, D), F32),
        scratch_types=[
            pltpu.VMEM((n_per_w,), jnp.int32),
            pltpu.VMEM((n_per_w, D), F32),
            pltpu.SemaphoreType.DMA,
        ],
    )
    def gather_k(table_hbm, idx_hbm, out_hbm, idx_v, rows_v, sem):
        wid = lax.axis_index("s") * info.num_cores + lax.axis_index("c")
        base = wid * n_per_w
        pltpu.sync_copy(idx_hbm.at[pl.ds(base, n_per_w)], idx_v)
        pltpu.async_copy(table_hbm.at[idx_v], rows_v, sem).wait()
        pltpu.sync_copy(rows_v, out_hbm.at[pl.ds(base, n_per_w)])

    return gather_k(codebook, idx)


def kernel(x, W_enc, b_enc, codebook, W_dec, b_dec):
    B, T, DIN = x.shape
    D = W_enc.shape[1]
    K = codebook.shape[0]
    N = B * T
    TM = 512
    x2 = x.reshape(N, DIN)

    cb_sq = pl.pallas_call(
        _cbsq_body,
        out_shape=jax.ShapeDtypeStruct((1, K), F32),
    )(codebook)

    z_ind3 = pl.pallas_call(
        _enc_vq_body,
        grid=(N // TM,),
        in_specs=[
            pl.BlockSpec((TM, DIN), lambda i: (i, 0)),
            pl.BlockSpec((DIN, D), lambda i: (0, 0)),
            pl.BlockSpec((1, D), lambda i: (0, 0)),
            pl.BlockSpec((K, D), lambda i: (0, 0)),
            pl.BlockSpec((1, K), lambda i: (0, 0)),
            pl.BlockSpec((1, K), lambda i: (0, 0)),
        ],
        out_specs=pl.BlockSpec((1, 1, TM), lambda i: (i, 0, 0)),
        out_shape=jax.ShapeDtypeStruct((N // TM, 1, TM), jnp.int32),
    )(x2, W_enc.astype(BF), b_enc.reshape(1, D), codebook.astype(BF), cb_sq,
      jnp.arange(K, dtype=F32).reshape(1, K))
    z_ind = z_ind3.reshape(N)

    z_q = _sc_gather(codebook, z_ind)

    TMD = 512
    x_rec = pl.pallas_call(
        _dec_body,
        grid=(N // TMD,),
        in_specs=[
            pl.BlockSpec((TMD, D), lambda i: (i, 0)),
            pl.BlockSpec((D, DIN), lambda i: (0, 0)),
            pl.BlockSpec((1, DIN), lambda i: (0, 0)),
        ],
        out_specs=pl.BlockSpec((TMD, DIN), lambda i: (i, 0)),
        out_shape=jax.ShapeDtypeStruct((N, DIN), F32),
    )(z_q, W_dec.astype(BF), b_dec.reshape(1, DIN))

    return (x_rec.reshape(B, T, DIN), z_q.reshape(B, T, D),
            z_ind.reshape(B, T))


# speedup vs baseline: 1.2413x; 1.0639x over previous
import functools

import jax
import jax.numpy as jnp
from jax import lax
from jax.experimental import pallas as pl
from jax.experimental.pallas import tpu as pltpu
from jax.experimental.pallas import tpu_sc as plsc

BF = jnp.bfloat16
F32 = jnp.float32


def _enc_vq_body(x_ref, w_ref, be_ref, cbf_ref, out_ref,
                 cbb_ref, cbsq_ref, kidx_ref, wb_ref):
    @pl.when(pl.program_id(0) == 0)
    def _():
        cbf = cbf_ref[...]
        cbb_ref[...] = cbf.astype(BF)
        cbsq_ref[...] = jnp.sum(cbf * cbf, axis=1).reshape(1, -1)
        kidx_ref[...] = lax.broadcasted_iota(
            jnp.int32, cbsq_ref.shape, 1).astype(F32)
        wb_ref[...] = w_ref[...].astype(BF)

    K = cbf_ref.shape[0]
    xb = x_ref[...].astype(BF)
    z_e = lax.dot_general(xb, wb_ref[...], (((1,), (0,)), ((), ())),
                          preferred_element_type=F32) + be_ref[...]
    ze_sq = jnp.sum(z_e * z_e, axis=1, keepdims=True)
    zeb = z_e.astype(BF)
    dots = lax.dot_general(zeb, cbb_ref[...], (((1,), (1,)), ((), ())),
                           preferred_element_type=F32)
    dist = ze_sq - 2.0 * dots + cbsq_ref[...]
    vmin = jnp.min(dist, axis=1, keepdims=True)
    cand = jnp.where(dist == vmin, kidx_ref[...], float(K))
    idxf = jnp.min(cand, axis=1)
    out_ref[...] = idxf.astype(jnp.int32).reshape(1, 1, -1)


def _dec_body(zq_ref, wd_ref, bd_ref, out_ref):
    zqb = zq_ref[...].astype(BF)
    wdb = wd_ref[...].astype(BF)
    out_ref[...] = lax.dot_general(
        zqb, wdb, (((1,), (0,)), ((), ())),
        preferred_element_type=F32) + bd_ref[...]


def _sc_gather(codebook, idx):
    K, D = codebook.shape
    N = idx.shape[0]
    info = plsc.get_sparse_core_info()
    nw = info.num_cores * info.num_subcores
    n_per_w = N // nw
    mesh = plsc.VectorSubcoreMesh(core_axis_name="c", subcore_axis_name="s")

    @functools.partial(
        pl.kernel, mesh=mesh,
        out_type=jax.ShapeDtypeStruct((N, D), F32),
        scratch_types=[
            pltpu.VMEM((n_per_w,), jnp.int32),
            pltpu.VMEM((n_per_w, D), F32),
            pltpu.SemaphoreType.DMA,
        ],
    )
    def gather_k(table_hbm, idx_hbm, out_hbm, idx_v, rows_v, sem):
        wid = lax.axis_index("s") * info.num_cores + lax.axis_index("c")
        base = wid * n_per_w
        pltpu.sync_copy(idx_hbm.at[pl.ds(base, n_per_w)], idx_v)
        pltpu.async_copy(table_hbm.at[idx_v], rows_v, sem).wait()
        pltpu.sync_copy(rows_v, out_hbm.at[pl.ds(base, n_per_w)])

    return gather_k(codebook, idx)


def kernel(x, W_enc, b_enc, codebook, W_dec, b_dec):
    B, T, DIN = x.shape
    D = W_enc.shape[1]
    K = codebook.shape[0]
    N = B * T
    TM = 512
    x2 = x.reshape(N, DIN)

    z_ind3 = pl.pallas_call(
        _enc_vq_body,
        grid=(N // TM,),
        in_specs=[
            pl.BlockSpec((TM, DIN), lambda i: (i, 0)),
            pl.BlockSpec((DIN, D), lambda i: (0, 0)),
            pl.BlockSpec((1, D), lambda i: (0, 0)),
            pl.BlockSpec((K, D), lambda i: (0, 0)),
        ],
        out_specs=pl.BlockSpec((1, 1, TM), lambda i: (i, 0, 0)),
        out_shape=jax.ShapeDtypeStruct((N // TM, 1, TM), jnp.int32),
        scratch_shapes=[
            pltpu.VMEM((K, D), BF),
            pltpu.VMEM((1, K), F32),
            pltpu.VMEM((1, K), F32),
            pltpu.VMEM((DIN, D), BF),
        ],
    )(x2, W_enc, b_enc.reshape(1, D), codebook)
    z_ind = z_ind3.reshape(N)

    z_q = _sc_gather(codebook, z_ind)

    TMD = 512
    x_rec = pl.pallas_call(
        _dec_body,
        grid=(N // TMD,),
        in_specs=[
            pl.BlockSpec((TMD, D), lambda i: (i, 0)),
            pl.BlockSpec((D, DIN), lambda i: (0, 0)),
            pl.BlockSpec((1, DIN), lambda i: (0, 0)),
        ],
        out_specs=pl.BlockSpec((TMD, DIN), lambda i: (i, 0)),
        out_shape=jax.ShapeDtypeStruct((N, DIN), F32),
    )(z_q, W_dec, b_dec.reshape(1, DIN))

    return (x_rec.reshape(B, T, DIN), z_q.reshape(B, T, D),
            z_ind.reshape(B, T))
